# baseline (speedup 1.0000x reference)
import jax
import jax.numpy as jnp
from jax import lax
from jax.experimental import pallas as pl
from jax.experimental.pallas import tpu as pltpu
from jax.experimental.pallas import tpu_sc as plsc

_B = 16384
_D = 64
_NC = 2
_NS = 16
_NW = _NC * _NS
_BPW = _B // _NW
_CHUNK = 64
_NCHUNK = _BPW // _CHUNK
_L = 16
_GPC = _CHUNK // _L
_PADN = 100096


def _rot(x, k):
    perm = (lax.iota(jnp.int32, _L) + k) & (_L - 1)
    return jnp.take_along_axis(x, perm, axis=0, mode="promise_in_bounds")


def _merge(a, b, block):
    half = block // 2
    first = (lax.iota(jnp.int32, _L) % block) < half
    m1 = jnp.where(first, a, _rot(b, -half))
    m2 = jnp.where(first, _rot(a, half), b)
    return m1 + m2


def _lane_sums(v):
    y = [_merge(v[i], v[i + 8], 16) for i in range(8)]
    z = [_merge(y[i], y[i + 4], 8) for i in range(4)]
    w = [_merge(z[i], z[i + 2], 4) for i in range(2)]
    return _merge(w[0], w[1], 2)


_HALF = 51200


def _widen_body(a_ref, b_ref, o_ref):
    o_ref[:, :_D] = jnp.transpose(a_ref[...], (1, 0))
    o_ref[:, _D:] = jnp.transpose(b_ref[...], (1, 0))


def _widen(table_t):
    d, n = table_t.shape
    blk = 12800
    nb = _HALF // blk
    return pl.pallas_call(
        _widen_body,
        grid=(nb,),
        in_specs=[pl.BlockSpec((d, blk), lambda i: (0, i)),
                  pl.BlockSpec((d, blk), lambda i: (0, i + nb))],
        out_specs=pl.BlockSpec((blk, 2 * d), lambda i: (i, 0)),
        out_shape=jax.ShapeDtypeStruct((_HALF, 2 * d), table_t.dtype),
        compiler_params=pltpu.CompilerParams(
            dimension_semantics=("arbitrary",)),
    )(table_t, table_t)


def _body(user_h, pos_h, neg_h, ut_h, it_h, pos_out, neg_out,
          uidx, pidx, nidx, uq, pq, nq, ubuf, pbuf, nbuf, psc, nsc,
          *sems):
    wid = lax.axis_index("s") * _NC + lax.axis_index("c")
    base = wid * _BPW

    pltpu.sync_copy(user_h.at[pl.ds(base, _BPW)], uidx)
    pltpu.sync_copy(pos_h.at[pl.ds(base, _BPW)], pidx)
    pltpu.sync_copy(neg_h.at[pl.ds(base, _BPW)], nidx)

    def halve(i, carry):
        s = pl.ds(i * _L, _L)
        u, p, n = uidx[s], pidx[s], nidx[s]
        uq[s] = jnp.where(u < _HALF, u, u - _HALF)
        pq[s] = jnp.where(p < _HALF, p, p - _HALF)
        nq[s] = jnp.where(n < _HALF, n, n - _HALF)
        return carry

    lax.fori_loop(0, _BPW // _L, halve, 0)

    def fire(c, buf_slot):
        s = pl.ds(c * _CHUNK, _CHUNK)
        return (
            pltpu.async_copy(ut_h.at[uq.at[s]], ubuf.at[buf_slot],
                             sems[buf_slot]),
            pltpu.async_copy(it_h.at[pq.at[s]], pbuf.at[buf_slot],
                             sems[2 + buf_slot]),
            pltpu.async_copy(it_h.at[nq.at[s]], nbuf.at[buf_slot],
                             sems[4 + buf_slot]),
        )

    def compute_chunk(c, buf_slot):
        ub, pb, nb = ubuf.at[buf_slot], pbuf.at[buf_slot], nbuf.at[buf_slot]

        def group(g, carry):
            gabs = c * _GPC + g
            gs = pl.ds(gabs * _L, _L)
            offu = jnp.where(uidx[gs] < _HALF, 0, _D).astype(jnp.int32)
            offp = jnp.where(pidx[gs] < _HALF, 0, _D).astype(jnp.int32)
            offn = jnp.where(nidx[gs] < _HALF, 0, _D).astype(jnp.int32)
            sp, sn = [], []
            for r in range(_L):
                slot = g * _L + r
                ou, op, on = offu[r], offp[r], offn[r]
                accp = None
                accn = None
                for k in range(4):
                    du = pl.multiple_of(ou + k * _L, _L)
                    dp = pl.multiple_of(op + k * _L, _L)
                    dn = pl.multiple_of(on + k * _L, _L)
                    u = ub[slot, pl.ds(du, _L)]
                    p = pb[slot, pl.ds(dp, _L)]
                    n = nb[slot, pl.ds(dn, _L)]
                    accp = u * p if accp is None else accp + u * p
                    accn = u * n if accn is None else accn + u * n
                sp.append(accp)
                sn.append(accn)
            psc[gs] = _lane_sums(sp)
            nsc[gs] = _lane_sums(sn)
            return carry

        lax.fori_loop(0, _GPC, group, 0)

    copies = [None] * _NCHUNK
    copies[0] = fire(0, 0)
    copies[1] = fire(1, 1)
    for c in range(_NCHUNK):
        for cp in copies[c]:
            cp.wait()
        compute_chunk(c, c % 2)
        if c + 2 < _NCHUNK:
            copies[c + 2] = fire(c + 2, c % 2)

    pltpu.sync_copy(psc, pos_out.at[pl.ds(base, _BPW)])
    pltpu.sync_copy(nsc, neg_out.at[pl.ds(base, _BPW)])


@jax.jit
def kernel(user, pos_item, neg_item, user_table, item_table):
    f32 = jnp.float32
    ut2 = _widen(jnp.swapaxes(user_table, 0, 1))
    it2 = _widen(jnp.swapaxes(item_table, 0, 1))
    run = pl.kernel(
        _body,
        out_type=[jax.ShapeDtypeStruct((_B,), f32),
                  jax.ShapeDtypeStruct((_B,), f32)],
        mesh=plsc.VectorSubcoreMesh(core_axis_name="c", subcore_axis_name="s"),
        compiler_params=pltpu.CompilerParams(needs_layout_passes=False),
        scratch_types=[
            pltpu.VMEM((_BPW,), jnp.int32),
            pltpu.VMEM((_BPW,), jnp.int32),
            pltpu.VMEM((_BPW,), jnp.int32),
            pltpu.VMEM((_BPW,), jnp.int32),
            pltpu.VMEM((_BPW,), jnp.int32),
            pltpu.VMEM((_BPW,), jnp.int32),
            pltpu.VMEM((2, _CHUNK, 2 * _D), f32),
            pltpu.VMEM((2, _CHUNK, 2 * _D), f32),
            pltpu.VMEM((2, _CHUNK, 2 * _D), f32),
            pltpu.VMEM((_BPW,), f32),
            pltpu.VMEM((_BPW,), f32),
        ] + [pltpu.SemaphoreType.DMA] * 6,
    )
    pos_score, neg_score = run(user.astype(jnp.int32),
                               pos_item.astype(jnp.int32),
                               neg_item.astype(jnp.int32),
                               ut2, it2)
    return (pos_score, neg_score)


# speedup vs baseline: 1.1059x; 1.1059x over previous
import jax
import jax.numpy as jnp
from jax import lax
from jax.experimental import pallas as pl
from jax.experimental.pallas import tpu as pltpu
from jax.experimental.pallas import tpu_sc as plsc

_B = 16384
_D = 64
_NC = 2
_NS = 16
_NW = _NC * _NS
_BPW = _B // _NW
_CHUNK = 128
_NCHUNK = _BPW // _CHUNK
_L = 16
_GPC = _CHUNK // _L
_PADN = 100096


def _rot(x, k):
    perm = (lax.iota(jnp.int32, _L) + k) & (_L - 1)
    return jnp.take_along_axis(x, perm, axis=0, mode="promise_in_bounds")


def _merge(a, b, block):
    half = block // 2
    first = (lax.iota(jnp.int32, _L) % block) < half
    m1 = jnp.where(first, a, _rot(b, -half))
    m2 = jnp.where(first, _rot(a, half), b)
    return m1 + m2


def _lane_sums(v):
    y = [_merge(v[i], v[i + 8], 16) for i in range(8)]
    z = [_merge(y[i], y[i + 4], 8) for i in range(4)]
    w = [_merge(z[i], z[i + 2], 4) for i in range(2)]
    return _merge(w[0], w[1], 2)


_HALF = 51200


def _widen_body(au_ref, bu_ref, ai_ref, bi_ref, ou_ref, oi_ref):
    ou_ref[:, :_D] = jnp.transpose(au_ref[...], (1, 0))
    ou_ref[:, _D:] = jnp.transpose(bu_ref[...], (1, 0))
    oi_ref[:, :_D] = jnp.transpose(ai_ref[...], (1, 0))
    oi_ref[:, _D:] = jnp.transpose(bi_ref[...], (1, 0))


def _widen(user_t, item_t):
    d, n = user_t.shape
    blk = 6400
    nb = _HALF // blk
    return pl.pallas_call(
        _widen_body,
        grid=(nb,),
        in_specs=[pl.BlockSpec((d, blk), lambda i: (0, i)),
                  pl.BlockSpec((d, blk), lambda i: (0, i + nb)),
                  pl.BlockSpec((d, blk), lambda i: (0, i)),
                  pl.BlockSpec((d, blk), lambda i: (0, i + nb))],
        out_specs=[pl.BlockSpec((blk, 2 * d), lambda i: (i, 0)),
                   pl.BlockSpec((blk, 2 * d), lambda i: (i, 0))],
        out_shape=[jax.ShapeDtypeStruct((_HALF, 2 * d), jnp.float32),
                   jax.ShapeDtypeStruct((_HALF, 2 * d), jnp.float32)],
        compiler_params=pltpu.CompilerParams(
            dimension_semantics=("arbitrary",)),
    )(user_t, user_t, item_t, item_t)


def _body(user_h, pos_h, neg_h, ut_h, it_h, pos_out, neg_out,
          uidx, pidx, nidx, uq, pq, nq, ubuf, pbuf, nbuf, psc, nsc,
          *sems):
    wid = lax.axis_index("s") * _NC + lax.axis_index("c")
    base = wid * _BPW

    pltpu.sync_copy(user_h.at[pl.ds(base, _BPW)], uidx)
    pltpu.sync_copy(pos_h.at[pl.ds(base, _BPW)], pidx)
    pltpu.sync_copy(neg_h.at[pl.ds(base, _BPW)], nidx)

    def halve(i, carry):
        s = pl.ds(i * _L, _L)
        u, p, n = uidx[s], pidx[s], nidx[s]
        uq[s] = jnp.where(u < _HALF, u, u - _HALF)
        pq[s] = jnp.where(p < _HALF, p, p - _HALF)
        nq[s] = jnp.where(n < _HALF, n, n - _HALF)
        return carry

    lax.fori_loop(0, _BPW // _L, halve, 0)

    def fire(c, buf_slot):
        s = pl.ds(c * _CHUNK, _CHUNK)
        return (
            pltpu.async_copy(ut_h.at[uq.at[s]], ubuf.at[buf_slot],
                             sems[buf_slot]),
            pltpu.async_copy(it_h.at[pq.at[s]], pbuf.at[buf_slot],
                             sems[2 + buf_slot]),
            pltpu.async_copy(it_h.at[nq.at[s]], nbuf.at[buf_slot],
                             sems[4 + buf_slot]),
        )

    def compute_chunk(c, buf_slot):
        ub, pb, nb = ubuf.at[buf_slot], pbuf.at[buf_slot], nbuf.at[buf_slot]

        def group(g, carry):
            gabs = c * _GPC + g
            gs = pl.ds(gabs * _L, _L)
            offu = jnp.where(uidx[gs] < _HALF, 0, _D).astype(jnp.int32)
            offp = jnp.where(pidx[gs] < _HALF, 0, _D).astype(jnp.int32)
            offn = jnp.where(nidx[gs] < _HALF, 0, _D).astype(jnp.int32)
            sp, sn = [], []
            for r in range(_L):
                slot = g * _L + r
                ou, op, on = offu[r], offp[r], offn[r]
                accp = None
                accn = None
                for k in range(4):
                    du = pl.multiple_of(ou + k * _L, _L)
                    dp = pl.multiple_of(op + k * _L, _L)
                    dn = pl.multiple_of(on + k * _L, _L)
                    u = ub[slot, pl.ds(du, _L)]
                    p = pb[slot, pl.ds(dp, _L)]
                    n = nb[slot, pl.ds(dn, _L)]
                    accp = u * p if accp is None else accp + u * p
                    accn = u * n if accn is None else accn + u * n
                sp.append(accp)
                sn.append(accn)
            psc[gs] = _lane_sums(sp)
            nsc[gs] = _lane_sums(sn)
            return carry

        lax.fori_loop(0, _GPC, group, 0)

    copies = [None] * _NCHUNK
    copies[0] = fire(0, 0)
    copies[1] = fire(1, 1)
    for c in range(_NCHUNK):
        for cp in copies[c]:
            cp.wait()
        compute_chunk(c, c % 2)
        if c + 2 < _NCHUNK:
            copies[c + 2] = fire(c + 2, c % 2)

    pltpu.sync_copy(psc, pos_out.at[pl.ds(base, _BPW)])
    pltpu.sync_copy(nsc, neg_out.at[pl.ds(base, _BPW)])


@jax.jit
def kernel(user, pos_item, neg_item, user_table, item_table):
    f32 = jnp.float32
    ut2, it2 = _widen(jnp.swapaxes(user_table, 0, 1),
                      jnp.swapaxes(item_table, 0, 1))
    run = pl.kernel(
        _body,
        out_type=[jax.ShapeDtypeStruct((_B,), f32),
                  jax.ShapeDtypeStruct((_B,), f32)],
        mesh=plsc.VectorSubcoreMesh(core_axis_name="c", subcore_axis_name="s"),
        compiler_params=pltpu.CompilerParams(needs_layout_passes=False),
        scratch_types=[
            pltpu.VMEM((_BPW,), jnp.int32),
            pltpu.VMEM((_BPW,), jnp.int32),
            pltpu.VMEM((_BPW,), jnp.int32),
            pltpu.VMEM((_BPW,), jnp.int32),
            pltpu.VMEM((_BPW,), jnp.int32),
            pltpu.VMEM((_BPW,), jnp.int32),
            pltpu.VMEM((2, _CHUNK, 2 * _D), f32),
            pltpu.VMEM((2, _CHUNK, 2 * _D), f32),
            pltpu.VMEM((2, _CHUNK, 2 * _D), f32),
            pltpu.VMEM((_BPW,), f32),
            pltpu.VMEM((_BPW,), f32),
        ] + [pltpu.SemaphoreType.DMA] * 6,
    )
    pos_score, neg_score = run(user.astype(jnp.int32),
                               pos_item.astype(jnp.int32),
                               neg_item.astype(jnp.int32),
                               ut2, it2)
    return (pos_score, neg_score)
